# baseline (speedup 1.0000x reference)
import functools

import numpy as np

import jax
import jax.numpy as jnp
from jax import lax
from jax.experimental import pallas as pl
from jax.experimental.pallas import tpu as pltpu
from jax.experimental.pallas import tpu_sc as plsc

NUM_FIELDS = 26
FIELD_SIZE = 100000
EMBED = 16
LANES = 16
NUM_WORKERS = 32
GSZ = 16 * NUM_FIELDS


def _make_kernel(batch: int, n_rows: int):
    per_w = n_rows // NUM_WORKERS
    n_groups = per_w // GSZ
    pairs = n_groups // 2
    mesh = plsc.VectorSubcoreMesh(core_axis_name="c", subcore_axis_name="s")

    @functools.partial(
        pl.kernel,
        out_type=jax.ShapeDtypeStruct((n_rows, EMBED), jnp.float32),
        mesh=mesh,
        compiler_params=pltpu.CompilerParams(
            use_tc_tiling_on_sc=False, needs_layout_passes=False),
        scratch_types=[
            pltpu.VMEM((per_w,), jnp.int32),
            pltpu.VMEM((GSZ,), jnp.int32),
            pltpu.VMEM((GSZ, EMBED), jnp.float32),
            pltpu.VMEM((GSZ, EMBED), jnp.float32),
            pltpu.SemaphoreType.DMA,
            pltpu.SemaphoreType.DMA,
            pltpu.SemaphoreType.DMA,
            pltpu.SemaphoreType.DMA,
        ],
    )
    def run(x_hbm, off_hbm, table_hbm, out_hbm, idx_v, off_v, buf_a, buf_b,
            gsem_a, gsem_b, ssem_a, ssem_b):
        wid = lax.axis_index("s") * 2 + lax.axis_index("c")
        base = wid * per_w
        pltpu.sync_copy(x_hbm.at[pl.ds(base, per_w)], idx_v)
        pltpu.sync_copy(off_hbm, off_v)

        def prep(g):
            for v in range(GSZ // LANES):
                off = pl.multiple_of(g * GSZ + v * LANES, LANES)
                s = pl.multiple_of(v * LANES, LANES)
                idx_v[pl.ds(off, LANES)] = (
                    idx_v[pl.ds(off, LANES)] + off_v[pl.ds(s, LANES)]
                )

        def fire_gather(g, buf, sem):
            for b in range(4):
                off = pl.multiple_of(g * GSZ + b * (GSZ // 4), 8)
                pltpu.async_copy(
                    table_hbm.at[idx_v.at[pl.ds(off, GSZ // 4)]],
                    buf.at[pl.ds(b * (GSZ // 4), GSZ // 4)], sem
                )

        def fire_store(g, buf, sem):
            row = pl.multiple_of(base + g * GSZ, 8)
            pltpu.async_copy(buf, out_hbm.at[pl.ds(row, GSZ)], sem)

        def drain_g(sem):
            for _ in range(4):
                pltpu.make_async_copy(
                    table_hbm.at[idx_v.at[pl.ds(0, GSZ // 4)]],
                    buf_a.at[pl.ds(0, GSZ // 4)], sem
                ).wait()

        def drain_s(sem):
            pltpu.make_async_copy(
                buf_a, out_hbm.at[pl.ds(base, GSZ)], sem
            ).wait()

        prep(0)
        fire_gather(0, buf_a, gsem_a)
        prep(1)
        fire_gather(1, buf_b, gsem_b)
        drain_g(gsem_a)
        fire_store(0, buf_a, ssem_a)

        def body(t, _):
            g0 = pl.multiple_of(2 * t, 2)
            g1 = g0 + 1
            prep(g0)
            drain_s(ssem_a)
            fire_gather(g0, buf_a, gsem_a)
            drain_g(gsem_b)
            fire_store(g1 - 2, buf_b, ssem_b)
            prep(g1)
            drain_s(ssem_b)
            fire_gather(g1, buf_b, gsem_b)
            drain_g(gsem_a)
            fire_store(g0, buf_a, ssem_a)
            return 0

        lax.fori_loop(1, pairs, body, 0)

        drain_g(gsem_b)
        fire_store(n_groups - 1, buf_b, ssem_b)
        drain_s(ssem_a)
        drain_s(ssem_b)

    return run


def kernel(x, table):
    batch, num_fields = x.shape
    n_rows = batch * num_fields
    x_flat = x.reshape(n_rows)
    off = jnp.asarray(
        (np.arange(GSZ, dtype=np.int32) % NUM_FIELDS) * FIELD_SIZE)
    out = _make_kernel(batch, n_rows)(x_flat, off, table)
    return out.reshape(batch, num_fields, EMBED)


# speedup vs baseline: 1.1542x; 1.1542x over previous
import functools

import jax
import jax.numpy as jnp
from jax import lax
from jax.experimental import pallas as pl
from jax.experimental.pallas import tpu as pltpu
from jax.experimental.pallas import tpu_sc as plsc

NUM_FIELDS = 26
FIELD_SIZE = 100000
EMBED = 16
LANES = 16
NUM_WORKERS = 32
BROWS = 4
CHUNK = BROWS * NUM_FIELDS
K = 4
GSZ = K * CHUNK


def _make_kernel(batch: int, n_rows: int):
    per_w = n_rows // NUM_WORKERS
    n_groups = per_w // GSZ
    pairs = n_groups // 2
    b_per_w = batch // NUM_WORKERS
    mesh = plsc.VectorSubcoreMesh(core_axis_name="c", subcore_axis_name="s")

    @functools.partial(
        pl.kernel,
        out_type=jax.ShapeDtypeStruct((batch, NUM_FIELDS, EMBED), jnp.float32),
        mesh=mesh,
        compiler_params=pltpu.CompilerParams(
            use_tc_tiling_on_sc=False, needs_layout_passes=False),
        scratch_types=[
            pltpu.VMEM((per_w,), jnp.int32),
            pltpu.VMEM((K, CHUNK, EMBED), jnp.float32),
            pltpu.VMEM((K, CHUNK, EMBED), jnp.float32),
            pltpu.SemaphoreType.DMA,
            pltpu.SemaphoreType.DMA,
            pltpu.SemaphoreType.DMA,
            pltpu.SemaphoreType.DMA,
        ],
    )
    def run(x_hbm, table_hbm, out_hbm, idx_v, buf_a, buf_b,
            gsem_a, gsem_b, ssem_a, ssem_b):
        wid = lax.axis_index("s") * 2 + lax.axis_index("c")
        base = wid * per_w
        brow0 = wid * b_per_w
        pltpu.sync_copy(x_hbm.at[pl.ds(base, per_w)], idx_v)

        lane = lax.broadcasted_iota(jnp.int32, (LANES,), 0)

        def prep(g):
            for v in range(GSZ // LANES):
                off = pl.multiple_of(g * GSZ + v * LANES, LANES)
                field = lax.rem(base + off + lane, NUM_FIELDS)
                idx_v[pl.ds(off, LANES)] = (
                    idx_v[pl.ds(off, LANES)] + field * FIELD_SIZE
                )

        def fire_gathers(g, buf, sem):
            for b in range(K):
                off = pl.multiple_of(g * GSZ + b * CHUNK, 8)
                pltpu.async_copy(
                    table_hbm.at[idx_v.at[pl.ds(off, CHUNK)]], buf.at[b], sem
                )

        def fire_stores(g, buf, sem):
            for b in range(K):
                row = pl.multiple_of(brow0 + g * (K * BROWS) + b * BROWS, BROWS)
                for r in range(BROWS):
                    pltpu.async_copy(
                        buf.at[b, pl.ds(r * NUM_FIELDS, NUM_FIELDS)],
                        out_hbm.at[row + r], sem
                    )

        def drain_g(sem, n):
            for _ in range(n):
                pltpu.make_async_copy(
                    table_hbm.at[idx_v.at[pl.ds(0, CHUNK)]], buf_a.at[0], sem
                ).wait()

        def drain_s(sem, n):
            for _ in range(n * BROWS):
                pltpu.make_async_copy(
                    buf_a.at[0, pl.ds(0, NUM_FIELDS)], out_hbm.at[brow0], sem
                ).wait()

        prep(0)
        fire_gathers(0, buf_a, gsem_a)
        prep(1)
        fire_gathers(1, buf_b, gsem_b)
        drain_g(gsem_a, K)
        fire_stores(0, buf_a, ssem_a)

        def body(t, _):
            g0 = pl.multiple_of(2 * t, 2)
            g1 = g0 + 1
            prep(g0)
            drain_s(ssem_a, K)
            fire_gathers(g0, buf_a, gsem_a)
            drain_g(gsem_b, K)
            fire_stores(g1 - 2, buf_b, ssem_b)
            prep(g1)
            drain_s(ssem_b, K)
            fire_gathers(g1, buf_b, gsem_b)
            drain_g(gsem_a, K)
            fire_stores(g0, buf_a, ssem_a)
            return 0

        lax.fori_loop(1, pairs, body, 0)

        drain_g(gsem_b, K)
        fire_stores(n_groups - 1, buf_b, ssem_b)
        drain_s(ssem_a, K)
        drain_s(ssem_b, K)

    return run


def kernel(x, table):
    batch, num_fields = x.shape
    n_rows = batch * num_fields
    x_flat = x.reshape(n_rows)
    return _make_kernel(batch, n_rows)(x_flat, table)
